# baseline (speedup 1.0000x reference)
import functools

import jax
import jax.numpy as jnp
from jax import lax
from jax.experimental import pallas as pl
from jax.experimental.pallas import tpu as pltpu
from jax.experimental.pallas import tpu_sc as plsc

_S, _X, _N, _D, _E, _B = 6000, 4000, 10000, 128, 320000, 4096
_NC, _NS = 2, 16
_CHUNK = 128
_NCHUNK = 160
_EPAD = _NS * _NCHUNK * _CHUNK
_ACC_N = _N + 16
_RPT = 624
_BN = 2000
_BPT = _B // (_NC * _NS)

_sc_mesh = plsc.VectorSubcoreMesh(core_axis_name="c", subcore_axis_name="s")


def _mm0_body(aw_ref, wr_ref, ww_ref, cjr_ref, cjw_ref, hr_ref, hw_ref):
    aw = aw_ref[...]
    hr_ref[...] = jnp.dot(aw, wr_ref[...], preferred_element_type=jnp.float32,
                          precision=lax.Precision.HIGHEST) * cjr_ref[...]
    hw_ref[...] = jnp.dot(aw, ww_ref[...], preferred_element_type=jnp.float32,
                          precision=lax.Precision.HIGHEST) * cjw_ref[...]


def _mm_first(aw, wr, ww, cjr, cjw):
    grid = (_N // _BN,)
    row = pl.BlockSpec((_BN, _D), lambda i: (i, 0))
    col = pl.BlockSpec((_BN, 1), lambda i: (i, 0))
    full = pl.BlockSpec((_D, _D), lambda i: (0, 0))
    return pl.pallas_call(
        _mm0_body,
        grid=grid,
        in_specs=[row, full, full, col, col],
        out_specs=[row, row],
        out_shape=[jax.ShapeDtypeStruct((_N, _D), jnp.float32)] * 2,
    )(aw, wr, ww, cjr, cjw)


def _mml_body(pr_ref, pw_ref, cir_ref, ciw_ref, wr_ref, ww_ref, cjr_ref,
              cjw_ref, hr_ref, hw_ref):
    xr = pr_ref[...] * cir_ref[...]
    xw = pw_ref[...] * ciw_ref[...]
    hr_ref[...] = jnp.dot(xr, wr_ref[...], preferred_element_type=jnp.float32,
                          precision=lax.Precision.HIGHEST) * cjr_ref[...]
    hw_ref[...] = jnp.dot(xw, ww_ref[...], preferred_element_type=jnp.float32,
                          precision=lax.Precision.HIGHEST) * cjw_ref[...]


def _mm_layer(pr, pw, cir, ciw, wr, ww, cjr, cjw):
    grid = (_N // _BN,)
    row = pl.BlockSpec((_BN, _D), lambda i: (i, 0))
    col = pl.BlockSpec((_BN, 1), lambda i: (i, 0))
    full = pl.BlockSpec((_D, _D), lambda i: (0, 0))
    return pl.pallas_call(
        _mml_body,
        grid=grid,
        in_specs=[row, row, col, col, full, full, col, col],
        out_specs=[row, row],
        out_shape=[jax.ShapeDtypeStruct((_N, _D), jnp.float32)] * 2,
    )(pr, pw, cir, ciw, wr, ww, cjr, cjw)


def _comb_body(pr_ref, pw_ref, cir_ref, ciw_ref, d_ref, o_ref, d128_ref):
    o_ref[...] = pr_ref[...] * cir_ref[...] + pw_ref[...] * ciw_ref[...]
    d128_ref[...] = jnp.broadcast_to(d_ref[...], d128_ref.shape)


def _combine(pr, pw, cir, ciw, disc):
    grid = (_N // _BN,)
    bx = _X // (_N // _BN)
    row = pl.BlockSpec((_BN, _D), lambda i: (i, 0))
    col = pl.BlockSpec((_BN, 1), lambda i: (i, 0))
    drow = pl.BlockSpec((bx, _D), lambda i: (i, 0))
    dcol = pl.BlockSpec((bx, 1), lambda i: (i, 0))
    return pl.pallas_call(
        _comb_body,
        grid=grid,
        in_specs=[row, row, col, col, dcol],
        out_specs=[row, drow],
        out_shape=[jax.ShapeDtypeStruct((_N, _D), jnp.float32),
                   jax.ShapeDtypeStruct((_X, _D), jnp.float32)],
    )(pr, pw, cir, ciw, disc)


_NBUF = 2
_NIG = _NCHUNK // 16


def _agg_body(hr_hbm, hw_hbm, srcr_hbm, dstr_hbm, srcw_hbm, dstw_hbm,
              zeros_hbm, outr_hbm, outw_hbm, cidx_v, srcg_v, dstg_v,
              rows_v, acc_sh, isem, gsem, ssem):
    cid = lax.axis_index("c")
    sid = lax.axis_index("s")
    row0 = sid * _RPT

    pltpu.sync_copy(zeros_hbm.at[pl.ds(0, _RPT)], acc_sh.at[pl.ds(row0, _RPT)])

    @pl.when(sid == _NS - 1)
    def _():
        pltpu.sync_copy(zeros_hbm.at[pl.ds(0, _N - _NS * _RPT)],
                        acc_sh.at[pl.ds(_NS * _RPT, _N - _NS * _RPT)])

    plsc.subcore_barrier()

    def run_graph(h_hbm, src_hbm, dst_hbm):
        def cfill(t, c):
            cidx_v.at[t][...] = (sid * _NCHUNK + t * 16
                                 + lax.iota(jnp.int32, 16))
            return c

        lax.fori_loop(0, _NIG, cfill, 0)

        def ifetch(g):
            sl = lax.rem(g, 2)
            pltpu.async_copy(src_hbm.at[cidx_v.at[g]], srcg_v.at[sl], isem)
            pltpu.async_copy(dst_hbm.at[cidx_v.at[g]], dstg_v.at[sl], isem)

        ifetch(0)

        def group(g, carry):
            sl = lax.rem(g, 2)
            pltpu.make_async_copy(src_hbm.at[cidx_v.at[0]], srcg_v.at[0],
                                  isem).wait()
            pltpu.make_async_copy(dst_hbm.at[cidx_v.at[0]], dstg_v.at[0],
                                  isem).wait()

            @pl.when(g + 1 < _NIG)
            def _():
                ifetch(g + 1)

            pltpu.async_copy(h_hbm.at[srcg_v.at[sl].at[0]], rows_v.at[0], gsem)
            pltpu.async_copy(h_hbm.at[srcg_v.at[sl].at[1]], rows_v.at[1], gsem)

            def chunk(i, c):
                b = lax.rem(i, 2)
                pltpu.make_async_copy(h_hbm.at[pl.ds(0, _CHUNK)],
                                      rows_v.at[0], gsem).wait()

                @pl.when(i >= 1)
                def _():
                    pltpu.make_async_copy(h_hbm.at[pl.ds(0, _CHUNK)],
                                          rows_v.at[0], ssem).wait()

                    @pl.when(i + 1 < 16)
                    def _():
                        pltpu.async_copy(h_hbm.at[srcg_v.at[sl].at[i + 1]],
                                         rows_v.at[1 - b], gsem)

                pltpu.async_copy(rows_v.at[b], acc_sh.at[dstg_v.at[sl].at[i]],
                                 ssem, add=True)
                return c

            lax.fori_loop(0, 16, chunk, 0)
            pltpu.make_async_copy(h_hbm.at[pl.ds(0, _CHUNK)], rows_v.at[0],
                                  ssem).wait()
            return carry

        lax.fori_loop(0, _NIG, group, 0)

    @pl.when(cid == 0)
    def _():
        run_graph(hr_hbm, srcr_hbm, dstr_hbm)

    @pl.when(cid == 1)
    def _():
        run_graph(hw_hbm, srcw_hbm, dstw_hbm)

    plsc.subcore_barrier()

    def writeback(out_hbm):
        pltpu.sync_copy(acc_sh.at[pl.ds(row0, _RPT)],
                        out_hbm.at[pl.ds(row0, _RPT)])

        @pl.when(sid == _NS - 1)
        def _():
            pltpu.sync_copy(acc_sh.at[pl.ds(_NS * _RPT, _N - _NS * _RPT)],
                            out_hbm.at[pl.ds(_NS * _RPT, _N - _NS * _RPT)])

    @pl.when(cid == 0)
    def _():
        writeback(outr_hbm)

    @pl.when(cid == 1)
    def _():
        writeback(outw_hbm)


_agg = pl.kernel(
    _agg_body,
    out_type=[jax.ShapeDtypeStruct((_N, _D), jnp.float32)] * 2,
    mesh=_sc_mesh,
    scratch_types=[
        pltpu.VMEM((_NIG, 16), jnp.int32),
        pltpu.VMEM((2, 16, _CHUNK), jnp.int32),
        pltpu.VMEM((2, 16, _CHUNK), jnp.int32),
        pltpu.VMEM((_NBUF, _CHUNK, _D), jnp.float32),
        pltpu.VMEM_SHARED((_ACC_N, _D), jnp.float32),
        pltpu.SemaphoreType.DMA,
        pltpu.SemaphoreType.DMA,
        pltpu.SemaphoreType.DMA,
    ],
)


def _gather_body(final_hbm, disc128_hbm, sid_hbm, eid_hbm, ostu_hbm, oexe_hbm,
                 odisc_hbm, idx_v, eidx_v, rows_v, sem):
    wid = lax.axis_index("s") * _NC + lax.axis_index("c")
    base = wid * _BPT

    pltpu.sync_copy(sid_hbm.at[pl.ds(base, _BPT)], idx_v)
    pltpu.async_copy(final_hbm.at[idx_v], rows_v, sem).wait()
    pltpu.sync_copy(rows_v, ostu_hbm.at[pl.ds(base, _BPT)])

    pltpu.sync_copy(eid_hbm.at[pl.ds(base, _BPT)], eidx_v)
    pltpu.async_copy(disc128_hbm.at[eidx_v], rows_v, sem).wait()
    pltpu.sync_copy(rows_v, odisc_hbm.at[pl.ds(base, _BPT)])
    for k in range(_BPT // 16):
        sl = pl.ds(k * 16, 16)
        eidx_v[sl] = eidx_v[sl] + _S
    pltpu.async_copy(final_hbm.at[eidx_v], rows_v, sem).wait()
    pltpu.sync_copy(rows_v, oexe_hbm.at[pl.ds(base, _BPT)])


_gather = pl.kernel(
    _gather_body,
    out_type=[
        jax.ShapeDtypeStruct((_B, _D), jnp.float32),
        jax.ShapeDtypeStruct((_B, _D), jnp.float32),
        jax.ShapeDtypeStruct((_B, _D), jnp.float32),
    ],
    mesh=_sc_mesh,
    scratch_types=[
        pltpu.VMEM((_BPT,), jnp.int32),
        pltpu.VMEM((_BPT,), jnp.int32),
        pltpu.VMEM((_BPT, _D), jnp.float32),
        pltpu.SemaphoreType.DMA,
    ],
)


def kernel(stu_emb, exer_emb, kn_emb, disc_emb, W_right, W_wrong,
           cj_r, ci_r, cj_w, ci_w, q_mask,
           src_r, dst_r, src_w, dst_w, student_id, exercise_id):
    aw = jnp.concatenate([stu_emb, exer_emb], axis=0)

    def pad_edges(a, fill):
        a = a.astype(jnp.int32).reshape(_NS, _E // _NS)
        a = jnp.pad(a, ((0, 0), (0, _EPAD // _NS - _E // _NS)),
                    constant_values=fill)
        return a.reshape(_NS * _NCHUNK, _CHUNK)

    srcr = pad_edges(src_r, 0)
    dstr = pad_edges(dst_r, _N)
    srcw = pad_edges(src_w, 0)
    dstw = pad_edges(dst_w, _N)
    zeros = jnp.zeros((_RPT, _D), jnp.float32)

    hr, hw = _mm_first(aw, W_right[0], W_wrong[0], cj_r, cj_w)
    pr, pw = _agg(hr, hw, srcr, dstr, srcw, dstw, zeros)
    for l in (1, 2):
        hr, hw = _mm_layer(pr, pw, ci_r, ci_w, W_right[l], W_wrong[l],
                           cj_r, cj_w)
        pr, pw = _agg(hr, hw, srcr, dstr, srcw, dstw, zeros)

    final, disc128 = _combine(pr, pw, ci_r, ci_w, disc_emb)
    bstu, bexe, bdisc = _gather(final, disc128,
                                student_id.astype(jnp.int32),
                                exercise_id.astype(jnp.int32))
    return bstu, bexe, bdisc[:, :1], kn_emb


# speedup vs baseline: 1.0655x; 1.0655x over previous
import functools

import jax
import jax.numpy as jnp
from jax import lax
from jax.experimental import pallas as pl
from jax.experimental.pallas import tpu as pltpu
from jax.experimental.pallas import tpu_sc as plsc

_S, _X, _N, _D, _E, _B = 6000, 4000, 10000, 128, 320000, 4096
_NC, _NS = 2, 16
_CHUNK = 128
_NCHUNK = 160
_EPAD = _NS * _NCHUNK * _CHUNK
_ACC_N = _N + 16
_RPT = 624
_BN = 2000
_BPT = _B // (_NC * _NS)

_sc_mesh = plsc.VectorSubcoreMesh(core_axis_name="c", subcore_axis_name="s")


def _mm0_body(aw_ref, wr_ref, ww_ref, cjr_ref, cjw_ref, hr_ref, hw_ref):
    aw = aw_ref[...]
    hr_ref[...] = jnp.dot(aw, wr_ref[...], preferred_element_type=jnp.float32,
                          precision=lax.Precision.HIGHEST) * cjr_ref[...]
    hw_ref[...] = jnp.dot(aw, ww_ref[...], preferred_element_type=jnp.float32,
                          precision=lax.Precision.HIGHEST) * cjw_ref[...]


def _mm_first(aw, wr, ww, cjr, cjw):
    grid = (_N // _BN,)
    row = pl.BlockSpec((_BN, _D), lambda i: (i, 0))
    col = pl.BlockSpec((_BN, 1), lambda i: (i, 0))
    full = pl.BlockSpec((_D, _D), lambda i: (0, 0))
    return pl.pallas_call(
        _mm0_body,
        grid=grid,
        in_specs=[row, full, full, col, col],
        out_specs=[row, row],
        out_shape=[jax.ShapeDtypeStruct((_N, _D), jnp.float32)] * 2,
    )(aw, wr, ww, cjr, cjw)


def _mml_body(pr_ref, pw_ref, cir_ref, ciw_ref, wr_ref, ww_ref, cjr_ref,
              cjw_ref, hr_ref, hw_ref):
    xr = pr_ref[...] * cir_ref[...]
    xw = pw_ref[...] * ciw_ref[...]
    hr_ref[...] = jnp.dot(xr, wr_ref[...], preferred_element_type=jnp.float32,
                          precision=lax.Precision.HIGHEST) * cjr_ref[...]
    hw_ref[...] = jnp.dot(xw, ww_ref[...], preferred_element_type=jnp.float32,
                          precision=lax.Precision.HIGHEST) * cjw_ref[...]


def _mm_layer(pr, pw, cir, ciw, wr, ww, cjr, cjw):
    grid = (_N // _BN,)
    row = pl.BlockSpec((_BN, _D), lambda i: (i, 0))
    col = pl.BlockSpec((_BN, 1), lambda i: (i, 0))
    full = pl.BlockSpec((_D, _D), lambda i: (0, 0))
    return pl.pallas_call(
        _mml_body,
        grid=grid,
        in_specs=[row, row, col, col, full, full, col, col],
        out_specs=[row, row],
        out_shape=[jax.ShapeDtypeStruct((_N, _D), jnp.float32)] * 2,
    )(pr, pw, cir, ciw, wr, ww, cjr, cjw)


def _comb_body(pr_ref, pw_ref, cir_ref, ciw_ref, d_ref, o_ref, d128_ref):
    o_ref[...] = pr_ref[...] * cir_ref[...] + pw_ref[...] * ciw_ref[...]
    d128_ref[...] = jnp.broadcast_to(d_ref[...], d128_ref.shape)


def _combine(pr, pw, cir, ciw, disc):
    grid = (_N // _BN,)
    bx = _X // (_N // _BN)
    row = pl.BlockSpec((_BN, _D), lambda i: (i, 0))
    col = pl.BlockSpec((_BN, 1), lambda i: (i, 0))
    drow = pl.BlockSpec((bx, _D), lambda i: (i, 0))
    dcol = pl.BlockSpec((bx, 1), lambda i: (i, 0))
    return pl.pallas_call(
        _comb_body,
        grid=grid,
        in_specs=[row, row, col, col, dcol],
        out_specs=[row, drow],
        out_shape=[jax.ShapeDtypeStruct((_N, _D), jnp.float32),
                   jax.ShapeDtypeStruct((_X, _D), jnp.float32)],
    )(pr, pw, cir, ciw, disc)


_NBUF = 2
_NIG = _NCHUNK // 16


def _agg_body(hr_hbm, hw_hbm, srcr_hbm, dstr_hbm, srcw_hbm, dstw_hbm,
              zeros_hbm, outr_hbm, outw_hbm, cidx_v, srcg_v, dstg_v,
              rows_v, acc_sh, isem, gsem):
    cid = lax.axis_index("c")
    sid = lax.axis_index("s")
    row0 = sid * _RPT

    pltpu.sync_copy(zeros_hbm.at[pl.ds(0, _RPT)], acc_sh.at[pl.ds(row0, _RPT)])

    @pl.when(sid == _NS - 1)
    def _():
        pltpu.sync_copy(zeros_hbm.at[pl.ds(0, _ACC_N - _NS * _RPT)],
                        acc_sh.at[pl.ds(_NS * _RPT, _ACC_N - _NS * _RPT)])

    plsc.subcore_barrier()

    def run_graph(h_hbm, src_hbm, dst_hbm):
        def cfill(t, c):
            cidx_v.at[t][...] = (sid * _NCHUNK + t * 16
                                 + lax.iota(jnp.int32, 16))
            return c

        lax.fori_loop(0, _NIG, cfill, 0)

        def ifetch(g):
            sl = lax.rem(g, 2)
            pltpu.async_copy(src_hbm.at[cidx_v.at[g]], srcg_v.at[sl], isem)
            pltpu.async_copy(dst_hbm.at[cidx_v.at[g]], dstg_v.at[sl], isem)

        ifetch(0)

        def group(g, carry):
            sl = lax.rem(g, 2)
            pltpu.make_async_copy(src_hbm.at[cidx_v.at[0]], srcg_v.at[0],
                                  isem).wait()
            pltpu.make_async_copy(dst_hbm.at[cidx_v.at[0]], dstg_v.at[0],
                                  isem).wait()

            @pl.when(g + 1 < _NIG)
            def _():
                ifetch(g + 1)

            pltpu.async_copy(h_hbm.at[srcg_v.at[sl].at[0]], rows_v.at[0], gsem)
            pltpu.async_copy(h_hbm.at[srcg_v.at[sl].at[1]], rows_v.at[1], gsem)

            def chunk(i, c):
                b = lax.rem(i, 2)
                pltpu.make_async_copy(h_hbm.at[pl.ds(0, _CHUNK)],
                                      rows_v.at[0], gsem).wait()
                pltpu.sync_copy(rows_v.at[b], acc_sh.at[dstg_v.at[sl].at[i]],
                                add=True)

                @pl.when(i + 2 < 16)
                def _():
                    pltpu.async_copy(h_hbm.at[srcg_v.at[sl].at[i + 2]],
                                     rows_v.at[b], gsem)

                return c

            lax.fori_loop(0, 16, chunk, 0)
            return carry

        lax.fori_loop(0, _NIG, group, 0)

    @pl.when(cid == 0)
    def _():
        run_graph(hr_hbm, srcr_hbm, dstr_hbm)

    @pl.when(cid == 1)
    def _():
        run_graph(hw_hbm, srcw_hbm, dstw_hbm)

    plsc.subcore_barrier()

    def writeback(out_hbm):
        pltpu.sync_copy(acc_sh.at[pl.ds(row0, _RPT)],
                        out_hbm.at[pl.ds(row0, _RPT)])

        @pl.when(sid == _NS - 1)
        def _():
            pltpu.sync_copy(acc_sh.at[pl.ds(_NS * _RPT, _N - _NS * _RPT)],
                            out_hbm.at[pl.ds(_NS * _RPT, _N - _NS * _RPT)])

    @pl.when(cid == 0)
    def _():
        writeback(outr_hbm)

    @pl.when(cid == 1)
    def _():
        writeback(outw_hbm)


_agg = pl.kernel(
    _agg_body,
    out_type=[jax.ShapeDtypeStruct((_N, _D), jnp.float32)] * 2,
    mesh=_sc_mesh,
    scratch_types=[
        pltpu.VMEM((_NIG, 16), jnp.int32),
        pltpu.VMEM((2, 16, _CHUNK), jnp.int32),
        pltpu.VMEM((2, 16, _CHUNK), jnp.int32),
        pltpu.VMEM((_NBUF, _CHUNK, _D), jnp.float32),
        pltpu.VMEM_SHARED((_ACC_N, _D), jnp.float32),
        pltpu.SemaphoreType.DMA,
        pltpu.SemaphoreType.DMA,
    ],
)


def _gather_body(final_hbm, disc128_hbm, sid_hbm, eid_hbm, ostu_hbm, oexe_hbm,
                 odisc_hbm, idx_v, eidx_v, rows_v, sem):
    wid = lax.axis_index("s") * _NC + lax.axis_index("c")
    base = wid * _BPT

    pltpu.sync_copy(sid_hbm.at[pl.ds(base, _BPT)], idx_v)
    pltpu.async_copy(final_hbm.at[idx_v], rows_v, sem).wait()
    pltpu.sync_copy(rows_v, ostu_hbm.at[pl.ds(base, _BPT)])

    pltpu.sync_copy(eid_hbm.at[pl.ds(base, _BPT)], eidx_v)
    pltpu.async_copy(disc128_hbm.at[eidx_v], rows_v, sem).wait()
    pltpu.sync_copy(rows_v, odisc_hbm.at[pl.ds(base, _BPT)])
    for k in range(_BPT // 16):
        sl = pl.ds(k * 16, 16)
        eidx_v[sl] = eidx_v[sl] + _S
    pltpu.async_copy(final_hbm.at[eidx_v], rows_v, sem).wait()
    pltpu.sync_copy(rows_v, oexe_hbm.at[pl.ds(base, _BPT)])


_gather = pl.kernel(
    _gather_body,
    out_type=[
        jax.ShapeDtypeStruct((_B, _D), jnp.float32),
        jax.ShapeDtypeStruct((_B, _D), jnp.float32),
        jax.ShapeDtypeStruct((_B, _D), jnp.float32),
    ],
    mesh=_sc_mesh,
    scratch_types=[
        pltpu.VMEM((_BPT,), jnp.int32),
        pltpu.VMEM((_BPT,), jnp.int32),
        pltpu.VMEM((_BPT, _D), jnp.float32),
        pltpu.SemaphoreType.DMA,
    ],
)


def kernel(stu_emb, exer_emb, kn_emb, disc_emb, W_right, W_wrong,
           cj_r, ci_r, cj_w, ci_w, q_mask,
           src_r, dst_r, src_w, dst_w, student_id, exercise_id):
    aw = jnp.concatenate([stu_emb, exer_emb], axis=0)

    def pad_edges(a, fill):
        a = a.astype(jnp.int32).reshape(_NS, _E // _NS)
        a = jnp.pad(a, ((0, 0), (0, _EPAD // _NS - _E // _NS)),
                    constant_values=fill)
        return a.reshape(_NS * _NCHUNK, _CHUNK)

    srcr = pad_edges(src_r, 0)
    dstr = pad_edges(dst_r, _N)
    srcw = pad_edges(src_w, 0)
    dstw = pad_edges(dst_w, _N)
    zeros = jnp.zeros((_RPT, _D), jnp.float32)

    hr, hw = _mm_first(aw, W_right[0], W_wrong[0], cj_r, cj_w)
    pr, pw = _agg(hr, hw, srcr, dstr, srcw, dstw, zeros)
    for l in (1, 2):
        hr, hw = _mm_layer(pr, pw, ci_r, ci_w, W_right[l], W_wrong[l],
                           cj_r, cj_w)
        pr, pw = _agg(hr, hw, srcr, dstr, srcw, dstw, zeros)

    final, disc128 = _combine(pr, pw, ci_r, ci_w, disc_emb)
    bstu, bexe, bdisc = _gather(final, disc128,
                                student_id.astype(jnp.int32),
                                exercise_id.astype(jnp.int32))
    return bstu, bexe, bdisc[:, :1], kn_emb


# speedup vs baseline: 1.0856x; 1.0188x over previous
import functools

import jax
import jax.numpy as jnp
from jax import lax
from jax.experimental import pallas as pl
from jax.experimental.pallas import tpu as pltpu
from jax.experimental.pallas import tpu_sc as plsc

_S, _X, _N, _D, _E, _B = 6000, 4000, 10000, 128, 320000, 4096
_NC, _NS = 2, 16
_CHUNK = 128
_NCHUNK = 160
_EPAD = _NS * _NCHUNK * _CHUNK
_ACC_N = _N + 16
_RPT = 624
_BN = 2000
_BPT = _B // (_NC * _NS)

_sc_mesh = plsc.VectorSubcoreMesh(core_axis_name="c", subcore_axis_name="s")


def _mm0_body(aw_ref, wr_ref, ww_ref, cjr_ref, cjw_ref, hr_ref, hw_ref):
    aw = aw_ref[...]
    hr_ref[...] = jnp.dot(aw, wr_ref[...], preferred_element_type=jnp.float32,
                          precision=lax.Precision.HIGHEST) * cjr_ref[...]
    hw_ref[...] = jnp.dot(aw, ww_ref[...], preferred_element_type=jnp.float32,
                          precision=lax.Precision.HIGHEST) * cjw_ref[...]


def _mm_first(aw, wr, ww, cjr, cjw):
    grid = (_N // _BN,)
    row = pl.BlockSpec((_BN, _D), lambda i: (i, 0))
    col = pl.BlockSpec((_BN, 1), lambda i: (i, 0))
    full = pl.BlockSpec((_D, _D), lambda i: (0, 0))
    return pl.pallas_call(
        _mm0_body,
        grid=grid,
        in_specs=[row, full, full, col, col],
        out_specs=[row, row],
        out_shape=[jax.ShapeDtypeStruct((_N, _D), jnp.float32)] * 2,
    )(aw, wr, ww, cjr, cjw)


def _mml_body(pr_ref, pw_ref, cir_ref, ciw_ref, wr_ref, ww_ref, cjr_ref,
              cjw_ref, hr_ref, hw_ref):
    xr = pr_ref[...] * cir_ref[...]
    xw = pw_ref[...] * ciw_ref[...]
    hr_ref[...] = jnp.dot(xr, wr_ref[...], preferred_element_type=jnp.float32,
                          precision=lax.Precision.HIGHEST) * cjr_ref[...]
    hw_ref[...] = jnp.dot(xw, ww_ref[...], preferred_element_type=jnp.float32,
                          precision=lax.Precision.HIGHEST) * cjw_ref[...]


def _mm_layer(pr, pw, cir, ciw, wr, ww, cjr, cjw):
    grid = (_N // _BN,)
    row = pl.BlockSpec((_BN, _D), lambda i: (i, 0))
    col = pl.BlockSpec((_BN, 1), lambda i: (i, 0))
    full = pl.BlockSpec((_D, _D), lambda i: (0, 0))
    return pl.pallas_call(
        _mml_body,
        grid=grid,
        in_specs=[row, row, col, col, full, full, col, col],
        out_specs=[row, row],
        out_shape=[jax.ShapeDtypeStruct((_N, _D), jnp.float32)] * 2,
    )(pr, pw, cir, ciw, wr, ww, cjr, cjw)


def _comb_body(pr_ref, pw_ref, cir_ref, ciw_ref, d_ref, o_ref, d128_ref):
    o_ref[...] = pr_ref[...] * cir_ref[...] + pw_ref[...] * ciw_ref[...]
    d128_ref[...] = jnp.broadcast_to(d_ref[...], d128_ref.shape)


def _combine(pr, pw, cir, ciw, disc):
    grid = (_N // _BN,)
    bx = _X // (_N // _BN)
    row = pl.BlockSpec((_BN, _D), lambda i: (i, 0))
    col = pl.BlockSpec((_BN, 1), lambda i: (i, 0))
    drow = pl.BlockSpec((bx, _D), lambda i: (i, 0))
    dcol = pl.BlockSpec((bx, 1), lambda i: (i, 0))
    return pl.pallas_call(
        _comb_body,
        grid=grid,
        in_specs=[row, row, col, col, dcol],
        out_specs=[row, drow],
        out_shape=[jax.ShapeDtypeStruct((_N, _D), jnp.float32),
                   jax.ShapeDtypeStruct((_X, _D), jnp.float32)],
    )(pr, pw, cir, ciw, disc)


_NBUF = 2
_NIG = _NCHUNK // 16


def _agg_body(hr_hbm, hw_hbm, srcr_hbm, dstr_hbm, srcw_hbm, dstw_hbm,
              zeros_hbm, outr_hbm, outw_hbm, cidx_v, srcg_v, dstg_v,
              rows_v, acc_sh, isem, gsem):
    cid = lax.axis_index("c")
    sid = lax.axis_index("s")
    row0 = sid * _RPT

    pltpu.sync_copy(zeros_hbm.at[pl.ds(0, _RPT)], acc_sh.at[pl.ds(row0, _RPT)])

    @pl.when(sid == _NS - 1)
    def _():
        pltpu.sync_copy(zeros_hbm.at[pl.ds(0, _ACC_N - _NS * _RPT)],
                        acc_sh.at[pl.ds(_NS * _RPT, _ACC_N - _NS * _RPT)])

    plsc.subcore_barrier()

    def run_graph(h_hbm, src_hbm, dst_hbm):
        def cfill(t, c):
            cidx_v.at[t][...] = (sid * _NCHUNK + t * 16
                                 + lax.iota(jnp.int32, 16))
            return c

        lax.fori_loop(0, _NIG, cfill, 0)

        def ifetch(g):
            sl = lax.rem(g, 2)
            pltpu.async_copy(src_hbm.at[cidx_v.at[g]], srcg_v.at[sl], isem)
            pltpu.async_copy(dst_hbm.at[cidx_v.at[g]], dstg_v.at[sl], isem)

        def iwait(src_hbm2, dst_hbm2):
            pltpu.make_async_copy(src_hbm2.at[cidx_v.at[0]], srcg_v.at[0],
                                  isem).wait()
            pltpu.make_async_copy(dst_hbm2.at[cidx_v.at[0]], dstg_v.at[0],
                                  isem).wait()

        ifetch(0)
        iwait(src_hbm, dst_hbm)
        ifetch(1)

        pltpu.async_copy(h_hbm.at[srcg_v.at[0].at[0]], rows_v.at[0], gsem)
        pltpu.async_copy(h_hbm.at[srcg_v.at[0].at[1]], rows_v.at[1], gsem)

        def chunk(j, c):
            g = lax.div(j, 16)
            i = lax.rem(j, 16)
            sl = lax.rem(g, 2)
            b = lax.rem(j, 2)
            pltpu.make_async_copy(h_hbm.at[pl.ds(0, _CHUNK)],
                                  rows_v.at[0], gsem).wait()
            pltpu.sync_copy(rows_v.at[b], acc_sh.at[dstg_v.at[sl].at[i]],
                            add=True)

            @pl.when(jnp.logical_and(i == 14, g + 1 < _NIG))
            def _():
                iwait(src_hbm, dst_hbm)

            nj = j + 2

            @pl.when(nj < _NCHUNK)
            def _():
                g2 = lax.div(nj, 16)
                pltpu.async_copy(
                    h_hbm.at[srcg_v.at[lax.rem(g2, 2)].at[lax.rem(nj, 16)]],
                    rows_v.at[b], gsem)

            @pl.when(jnp.logical_and(i == 15, g + 2 < _NIG))
            def _():
                ifetch(g + 2)

            return c

        lax.fori_loop(0, _NCHUNK, chunk, 0)

    @pl.when(cid == 0)
    def _():
        run_graph(hr_hbm, srcr_hbm, dstr_hbm)

    @pl.when(cid == 1)
    def _():
        run_graph(hw_hbm, srcw_hbm, dstw_hbm)

    plsc.subcore_barrier()

    def writeback(out_hbm):
        pltpu.sync_copy(acc_sh.at[pl.ds(row0, _RPT)],
                        out_hbm.at[pl.ds(row0, _RPT)])

        @pl.when(sid == _NS - 1)
        def _():
            pltpu.sync_copy(acc_sh.at[pl.ds(_NS * _RPT, _N - _NS * _RPT)],
                            out_hbm.at[pl.ds(_NS * _RPT, _N - _NS * _RPT)])

    @pl.when(cid == 0)
    def _():
        writeback(outr_hbm)

    @pl.when(cid == 1)
    def _():
        writeback(outw_hbm)


_agg = pl.kernel(
    _agg_body,
    out_type=[jax.ShapeDtypeStruct((_N, _D), jnp.float32)] * 2,
    mesh=_sc_mesh,
    scratch_types=[
        pltpu.VMEM((_NIG, 16), jnp.int32),
        pltpu.VMEM((2, 16, _CHUNK), jnp.int32),
        pltpu.VMEM((2, 16, _CHUNK), jnp.int32),
        pltpu.VMEM((_NBUF, _CHUNK, _D), jnp.float32),
        pltpu.VMEM_SHARED((_ACC_N, _D), jnp.float32),
        pltpu.SemaphoreType.DMA,
        pltpu.SemaphoreType.DMA,
    ],
)


def _gather_body(final_hbm, disc128_hbm, sid_hbm, eid_hbm, ostu_hbm, oexe_hbm,
                 odisc_hbm, idx_v, eidx_v, rows_v, sem):
    wid = lax.axis_index("s") * _NC + lax.axis_index("c")
    base = wid * _BPT

    pltpu.sync_copy(sid_hbm.at[pl.ds(base, _BPT)], idx_v)
    pltpu.async_copy(final_hbm.at[idx_v], rows_v, sem).wait()
    pltpu.sync_copy(rows_v, ostu_hbm.at[pl.ds(base, _BPT)])

    pltpu.sync_copy(eid_hbm.at[pl.ds(base, _BPT)], eidx_v)
    pltpu.async_copy(disc128_hbm.at[eidx_v], rows_v, sem).wait()
    pltpu.sync_copy(rows_v, odisc_hbm.at[pl.ds(base, _BPT)])
    for k in range(_BPT // 16):
        sl = pl.ds(k * 16, 16)
        eidx_v[sl] = eidx_v[sl] + _S
    pltpu.async_copy(final_hbm.at[eidx_v], rows_v, sem).wait()
    pltpu.sync_copy(rows_v, oexe_hbm.at[pl.ds(base, _BPT)])


_gather = pl.kernel(
    _gather_body,
    out_type=[
        jax.ShapeDtypeStruct((_B, _D), jnp.float32),
        jax.ShapeDtypeStruct((_B, _D), jnp.float32),
        jax.ShapeDtypeStruct((_B, _D), jnp.float32),
    ],
    mesh=_sc_mesh,
    scratch_types=[
        pltpu.VMEM((_BPT,), jnp.int32),
        pltpu.VMEM((_BPT,), jnp.int32),
        pltpu.VMEM((_BPT, _D), jnp.float32),
        pltpu.SemaphoreType.DMA,
    ],
)


def kernel(stu_emb, exer_emb, kn_emb, disc_emb, W_right, W_wrong,
           cj_r, ci_r, cj_w, ci_w, q_mask,
           src_r, dst_r, src_w, dst_w, student_id, exercise_id):
    aw = jnp.concatenate([stu_emb, exer_emb], axis=0)

    def pad_edges(a, fill):
        a = a.astype(jnp.int32).reshape(_NS, _E // _NS)
        a = jnp.pad(a, ((0, 0), (0, _EPAD // _NS - _E // _NS)),
                    constant_values=fill)
        return a.reshape(_NS * _NCHUNK, _CHUNK)

    srcr = pad_edges(src_r, 0)
    dstr = pad_edges(dst_r, _N)
    srcw = pad_edges(src_w, 0)
    dstw = pad_edges(dst_w, _N)
    zeros = jnp.zeros((_RPT, _D), jnp.float32)

    hr, hw = _mm_first(aw, W_right[0], W_wrong[0], cj_r, cj_w)
    pr, pw = _agg(hr, hw, srcr, dstr, srcw, dstw, zeros)
    for l in (1, 2):
        hr, hw = _mm_layer(pr, pw, ci_r, ci_w, W_right[l], W_wrong[l],
                           cj_r, cj_w)
        pr, pw = _agg(hr, hw, srcr, dstr, srcw, dstw, zeros)

    final, disc128 = _combine(pr, pw, ci_r, ci_w, disc_emb)
    bstu, bexe, bdisc = _gather(final, disc128,
                                student_id.astype(jnp.int32),
                                exercise_id.astype(jnp.int32))
    return bstu, bexe, bdisc[:, :1], kn_emb
